# baseline (speedup 1.0000x reference)
import functools

import jax
import jax.numpy as jnp
from jax import lax
from jax.experimental import pallas as pl
from jax.experimental.pallas import tpu as pltpu
from jax.experimental.pallas import tpu_sc as plsc

N_NODES = 10000
N_EDGES = 320000
D = 128

NC = 2
NS = 16
NW = NC * NS

CHUNK = 64
CPT = 156
EPT = CHUNK * CPT
N_EXTRA = (N_EDGES - EPT * NW) // CHUNK
EXTRA_BASE = EPT * NW

NBUF = 5
LOOKAHEAD = 4
NI = 8
ILOOK = 5

STRIPE = 640
LAST_STRIPE = N_NODES - 15 * STRIPE

_mesh = plsc.VectorSubcoreMesh(core_axis_name="c", subcore_axis_name="s")


@functools.partial(
    pl.kernel,
    mesh=_mesh,
    out_type=jax.ShapeDtypeStruct((NC, N_NODES, D), jnp.float32),
    scratch_types=[
        pltpu.VMEM((NI, 2, CHUNK), jnp.int32),
        pltpu.VMEM((2, CHUNK), jnp.int32),
        pltpu.VMEM((NBUF, CHUNK, D), jnp.float32),
        pltpu.VMEM_SHARED((N_NODES, D), jnp.float32),
        pltpu.SemaphoreType.DMA((NI,)),
        pltpu.SemaphoreType.DMA,
        pltpu.SemaphoreType.DMA((NBUF,)),
        pltpu.SemaphoreType.DMA((NBUF,)),
    ],
)
def _sc_aggregate(feat_hbm, edge_hbm, out_hbm, ibuf, xbuf, rows, acc,
                  isem, xsem, gsem, asem):
    c = lax.axis_index("c")
    s = lax.axis_index("s")
    wid = s * NC + c

    @pl.when(wid < N_EXTRA)
    def _():
        base = EXTRA_BASE + wid * CHUNK
        pltpu.async_copy(edge_hbm.at[0, pl.ds(base, CHUNK)], xbuf.at[0], xsem)
        pltpu.async_copy(edge_hbm.at[1, pl.ds(base, CHUNK)], xbuf.at[1], xsem)

    @pl.when(s < 15)
    def _():
        sl = pl.ds(s * STRIPE, STRIPE)
        pltpu.sync_copy(feat_hbm.at[sl], acc.at[sl])

    @pl.when(s == 15)
    def _():
        sl = pl.ds(15 * STRIPE, LAST_STRIPE)
        pltpu.sync_copy(feat_hbm.at[sl], acc.at[sl])

    plsc.subcore_barrier()

    def ifetch(j):
        base = wid * EPT + j * CHUNK
        h0 = pltpu.async_copy(
            edge_hbm.at[0, pl.ds(base, CHUNK)], ibuf.at[j % NI, 0],
            isem.at[j % NI])
        h1 = pltpu.async_copy(
            edge_hbm.at[1, pl.ds(base, CHUNK)], ibuf.at[j % NI, 1],
            isem.at[j % NI])
        return h0, h1

    def gather_start(j):
        return pltpu.async_copy(
            feat_hbm.at[ibuf.at[j % NI, 0]], rows.at[j % NBUF],
            gsem.at[j % NBUF])

    def add_start(j):
        return pltpu.async_copy(
            rows.at[j % NBUF], acc.at[ibuf.at[j % NI, 1]],
            asem.at[j % NBUF], add=True)

    ih, gh, ah = {}, {}, {}
    for j in range(ILOOK):
        ih[j] = ifetch(j)
    for j in range(LOOKAHEAD):
        ih[j][0].wait()
        ih[j][1].wait()
        gh[j] = gather_start(j)
    for j in range(CPT):
        ji = j + ILOOK
        if ji < CPT:
            ih[ji] = ifetch(ji)
        jn = j + LOOKAHEAD
        if jn < CPT:
            if jn - NBUF >= 0:
                ah[jn - NBUF].wait()
            ih[jn][0].wait()
            ih[jn][1].wait()
            gh[jn] = gather_start(jn)
        gh[j].wait()
        ah[j] = add_start(j)
    for j in range(CPT - NBUF, CPT):
        ah[j].wait()

    @pl.when(wid < N_EXTRA)
    def _():
        pltpu.make_async_copy(
            edge_hbm.at[0, pl.ds(EXTRA_BASE, CHUNK)], xbuf.at[0], xsem).wait()
        pltpu.make_async_copy(
            edge_hbm.at[1, pl.ds(EXTRA_BASE, CHUNK)], xbuf.at[1], xsem).wait()
        pltpu.async_copy(
            feat_hbm.at[xbuf.at[0]], rows.at[0], gsem.at[0]).wait()
        pltpu.async_copy(
            rows.at[0], acc.at[xbuf.at[1]], asem.at[0], add=True).wait()

    plsc.subcore_barrier()

    @pl.when(s < 15)
    def _():
        sl = pl.ds(s * STRIPE, STRIPE)
        pltpu.sync_copy(acc.at[sl], out_hbm.at[c, sl])

    @pl.when(s == 15)
    def _():
        sl = pl.ds(15 * STRIPE, LAST_STRIPE)
        pltpu.sync_copy(acc.at[sl], out_hbm.at[c, sl])


def _combine_body(p_ref, f_ref, o_ref):
    o_ref[...] = p_ref[0] + p_ref[1] - f_ref[...]


_ROWS_BLK = 1000

_combine = pl.pallas_call(
    _combine_body,
    grid=(N_NODES // _ROWS_BLK,),
    in_specs=[
        pl.BlockSpec((NC, _ROWS_BLK, D), lambda i: (0, i, 0)),
        pl.BlockSpec((_ROWS_BLK, D), lambda i: (i, 0)),
    ],
    out_specs=pl.BlockSpec((_ROWS_BLK, D), lambda i: (i, 0)),
    out_shape=jax.ShapeDtypeStruct((N_NODES, D), jnp.float32),
)


def kernel(feat, edge_index):
    partial = _sc_aggregate(feat, edge_index.astype(jnp.int32))
    return _combine(partial, feat)


# speedup vs baseline: 1.0176x; 1.0176x over previous
import functools

import jax
import jax.numpy as jnp
from jax import lax
from jax.experimental import pallas as pl
from jax.experimental.pallas import tpu as pltpu
from jax.experimental.pallas import tpu_sc as plsc

N_NODES = 10000
N_EDGES = 320000
D = 128

NC = 2
NS = 16
NW = NC * NS

CHUNK = 128
CPT = 78
EPT = CHUNK * CPT
N_EXTRA = (N_EDGES - EPT * NW) // CHUNK
EXTRA_BASE = EPT * NW

NBUF = 3
LOOKAHEAD = 2
NI = 6
ILOOK = 3

STRIPE = 640
LAST_STRIPE = N_NODES - 15 * STRIPE

_mesh = plsc.VectorSubcoreMesh(core_axis_name="c", subcore_axis_name="s")


@functools.partial(
    pl.kernel,
    mesh=_mesh,
    out_type=jax.ShapeDtypeStruct((NC, N_NODES, D), jnp.float32),
    scratch_types=[
        pltpu.VMEM((NI, 2, CHUNK), jnp.int32),
        pltpu.VMEM((2, CHUNK), jnp.int32),
        pltpu.VMEM((NBUF, CHUNK, D), jnp.float32),
        pltpu.VMEM_SHARED((N_NODES, D), jnp.float32),
        pltpu.SemaphoreType.DMA((NI,)),
        pltpu.SemaphoreType.DMA,
        pltpu.SemaphoreType.DMA((NBUF,)),
        pltpu.SemaphoreType.DMA((NBUF,)),
    ],
)
def _sc_aggregate(feat_hbm, edge_hbm, out_hbm, ibuf, xbuf, rows, acc,
                  isem, xsem, gsem, asem):
    c = lax.axis_index("c")
    s = lax.axis_index("s")
    wid = s * NC + c

    @pl.when(wid < N_EXTRA)
    def _():
        base = EXTRA_BASE + wid * CHUNK
        pltpu.async_copy(edge_hbm.at[0, pl.ds(base, CHUNK)], xbuf.at[0], xsem)
        pltpu.async_copy(edge_hbm.at[1, pl.ds(base, CHUNK)], xbuf.at[1], xsem)

    @pl.when(s < 15)
    def _():
        sl = pl.ds(s * STRIPE, STRIPE)
        pltpu.sync_copy(feat_hbm.at[sl], acc.at[sl])

    @pl.when(s == 15)
    def _():
        sl = pl.ds(15 * STRIPE, LAST_STRIPE)
        pltpu.sync_copy(feat_hbm.at[sl], acc.at[sl])

    plsc.subcore_barrier()

    def ifetch(j):
        base = wid * EPT + j * CHUNK
        h0 = pltpu.async_copy(
            edge_hbm.at[0, pl.ds(base, CHUNK)], ibuf.at[j % NI, 0],
            isem.at[j % NI])
        h1 = pltpu.async_copy(
            edge_hbm.at[1, pl.ds(base, CHUNK)], ibuf.at[j % NI, 1],
            isem.at[j % NI])
        return h0, h1

    def gather_start(j):
        return pltpu.async_copy(
            feat_hbm.at[ibuf.at[j % NI, 0]], rows.at[j % NBUF],
            gsem.at[j % NBUF])

    def add_start(j):
        return pltpu.async_copy(
            rows.at[j % NBUF], acc.at[ibuf.at[j % NI, 1]],
            asem.at[j % NBUF], add=True)

    ih, gh, ah = {}, {}, {}
    for j in range(ILOOK):
        ih[j] = ifetch(j)
    for j in range(LOOKAHEAD):
        ih[j][0].wait()
        ih[j][1].wait()
        gh[j] = gather_start(j)
    for j in range(CPT):
        ji = j + ILOOK
        if ji < CPT:
            ih[ji] = ifetch(ji)
        jn = j + LOOKAHEAD
        if jn < CPT:
            if jn - NBUF >= 0:
                ah[jn - NBUF].wait()
            ih[jn][0].wait()
            ih[jn][1].wait()
            gh[jn] = gather_start(jn)
        gh[j].wait()
        ah[j] = add_start(j)
    for j in range(CPT - NBUF, CPT):
        ah[j].wait()

    @pl.when(wid < N_EXTRA)
    def _():
        pltpu.make_async_copy(
            edge_hbm.at[0, pl.ds(EXTRA_BASE, CHUNK)], xbuf.at[0], xsem).wait()
        pltpu.make_async_copy(
            edge_hbm.at[1, pl.ds(EXTRA_BASE, CHUNK)], xbuf.at[1], xsem).wait()
        pltpu.async_copy(
            feat_hbm.at[xbuf.at[0]], rows.at[0], gsem.at[0]).wait()
        pltpu.async_copy(
            rows.at[0], acc.at[xbuf.at[1]], asem.at[0], add=True).wait()

    plsc.subcore_barrier()

    @pl.when(s < 15)
    def _():
        sl = pl.ds(s * STRIPE, STRIPE)
        pltpu.sync_copy(acc.at[sl], out_hbm.at[c, sl])

    @pl.when(s == 15)
    def _():
        sl = pl.ds(15 * STRIPE, LAST_STRIPE)
        pltpu.sync_copy(acc.at[sl], out_hbm.at[c, sl])


def _combine_body(p_ref, f_ref, o_ref):
    o_ref[...] = p_ref[0] + p_ref[1] - f_ref[...]


_ROWS_BLK = 1000

_combine = pl.pallas_call(
    _combine_body,
    grid=(N_NODES // _ROWS_BLK,),
    in_specs=[
        pl.BlockSpec((NC, _ROWS_BLK, D), lambda i: (0, i, 0)),
        pl.BlockSpec((_ROWS_BLK, D), lambda i: (i, 0)),
    ],
    out_specs=pl.BlockSpec((_ROWS_BLK, D), lambda i: (i, 0)),
    out_shape=jax.ShapeDtypeStruct((N_NODES, D), jnp.float32),
)


def kernel(feat, edge_index):
    partial = _sc_aggregate(feat, edge_index.astype(jnp.int32))
    return _combine(partial, feat)


# speedup vs baseline: 1.0366x; 1.0187x over previous
import functools

import jax
import jax.numpy as jnp
from jax import lax
from jax.experimental import pallas as pl
from jax.experimental.pallas import tpu as pltpu
from jax.experimental.pallas import tpu_sc as plsc

N_NODES = 10000
N_EDGES = 320000
D = 128

NC = 2
NS = 16
NW = NC * NS

CHUNK = 128
CPT = 78
EPT = CHUNK * CPT
N_EXTRA = (N_EDGES - EPT * NW) // CHUNK
EXTRA_BASE = EPT * NW

NBUF = 3
LOOKAHEAD = 2
NI = 6
ILOOK = 3

STRIPE = 640
LAST_STRIPE = N_NODES - 15 * STRIPE

_mesh = plsc.VectorSubcoreMesh(core_axis_name="c", subcore_axis_name="s")


@functools.partial(
    pl.kernel,
    mesh=_mesh,
    out_type=jax.ShapeDtypeStruct((NC, N_NODES, D), jnp.float32),
    scratch_types=[
        pltpu.VMEM((NI, 2, CHUNK), jnp.int32),
        pltpu.VMEM((2, CHUNK), jnp.int32),
        pltpu.VMEM((NBUF, CHUNK, D), jnp.float32),
        pltpu.VMEM_SHARED((N_NODES, D), jnp.float32),
        pltpu.SemaphoreType.DMA((NI,)),
        pltpu.SemaphoreType.DMA,
        pltpu.SemaphoreType.DMA((NBUF,)),
        pltpu.SemaphoreType.DMA((NBUF,)),
    ],
)
def _sc_aggregate(feat_hbm, edge_hbm, out_hbm, ibuf, xbuf, rows, acc,
                  isem, xsem, gsem, asem):
    c = lax.axis_index("c")
    s = lax.axis_index("s")
    wid = s * NC + c

    @pl.when(wid < N_EXTRA)
    def _():
        base = EXTRA_BASE + wid * CHUNK
        pltpu.async_copy(edge_hbm.at[0, pl.ds(base, CHUNK)], xbuf.at[0], xsem)
        pltpu.async_copy(edge_hbm.at[1, pl.ds(base, CHUNK)], xbuf.at[1], xsem)

    @pl.when(s < 15)
    def _():
        sl = pl.ds(s * STRIPE, STRIPE)
        pltpu.sync_copy(feat_hbm.at[sl], acc.at[sl])

    @pl.when(s == 15)
    def _():
        sl = pl.ds(15 * STRIPE, LAST_STRIPE)
        pltpu.sync_copy(feat_hbm.at[sl], acc.at[sl])

    plsc.subcore_barrier()

    def ifetch(j):
        base = wid * EPT + j * CHUNK
        h0 = pltpu.async_copy(
            edge_hbm.at[0, pl.ds(base, CHUNK)], ibuf.at[j % NI, 0],
            isem.at[j % NI])
        h1 = pltpu.async_copy(
            edge_hbm.at[1, pl.ds(base, CHUNK)], ibuf.at[j % NI, 1],
            isem.at[j % NI])
        return h0, h1

    def gather_start(j):
        return pltpu.async_copy(
            feat_hbm.at[ibuf.at[j % NI, 0]], rows.at[j % NBUF],
            gsem.at[j % NBUF])

    def add_start(j):
        return pltpu.async_copy(
            rows.at[j % NBUF], acc.at[ibuf.at[j % NI, 1]],
            asem.at[j % NBUF], add=True)

    ih, gh, ah = {}, {}, {}
    for j in range(ILOOK):
        ih[j] = ifetch(j)
    for j in range(LOOKAHEAD):
        ih[j][0].wait()
        ih[j][1].wait()
        gh[j] = gather_start(j)
    for j in range(CPT):
        ji = j + ILOOK
        if ji < CPT:
            ih[ji] = ifetch(ji)
        jn = j + LOOKAHEAD
        if jn < CPT:
            if jn - NBUF >= 0:
                ah[jn - NBUF].wait()
            ih[jn][0].wait()
            ih[jn][1].wait()
            gh[jn] = gather_start(jn)
        gh[j].wait()
        ah[j] = add_start(j)
    for j in range(CPT - NBUF, CPT):
        ah[j].wait()

    @pl.when(wid < N_EXTRA)
    def _():
        pltpu.make_async_copy(
            edge_hbm.at[0, pl.ds(EXTRA_BASE, CHUNK)], xbuf.at[0], xsem).wait()
        pltpu.make_async_copy(
            edge_hbm.at[1, pl.ds(EXTRA_BASE, CHUNK)], xbuf.at[1], xsem).wait()
        pltpu.async_copy(
            feat_hbm.at[xbuf.at[0]], rows.at[0], gsem.at[0]).wait()
        pltpu.async_copy(
            rows.at[0], acc.at[xbuf.at[1]], asem.at[0], add=True).wait()

    plsc.subcore_barrier()

    @pl.when(s < 15)
    def _():
        sl = pl.ds(s * STRIPE, STRIPE)
        pltpu.sync_copy(acc.at[sl], out_hbm.at[c, sl])

    @pl.when(s == 15)
    def _():
        sl = pl.ds(15 * STRIPE, LAST_STRIPE)
        pltpu.sync_copy(acc.at[sl], out_hbm.at[c, sl])


def _combine_body(p_ref, f_ref, o_ref):
    o_ref[...] = p_ref[0] + p_ref[1] - f_ref[...]


_ROWS_BLK = 2000

_combine = pl.pallas_call(
    _combine_body,
    grid=(N_NODES // _ROWS_BLK,),
    in_specs=[
        pl.BlockSpec((NC, _ROWS_BLK, D), lambda i: (0, i, 0)),
        pl.BlockSpec((_ROWS_BLK, D), lambda i: (i, 0)),
    ],
    out_specs=pl.BlockSpec((_ROWS_BLK, D), lambda i: (i, 0)),
    out_shape=jax.ShapeDtypeStruct((N_NODES, D), jnp.float32),
)


def kernel(feat, edge_index):
    partial = _sc_aggregate(feat, edge_index.astype(jnp.int32))
    return _combine(partial, feat)


# speedup vs baseline: 1.0413x; 1.0045x over previous
import functools

import jax
import jax.numpy as jnp
from jax import lax
from jax.experimental import pallas as pl
from jax.experimental.pallas import tpu as pltpu
from jax.experimental.pallas import tpu_sc as plsc

N_NODES = 10000
N_EDGES = 320000
D = 128

NC = 2
NS = 16
NW = NC * NS

CHUNK = 128
CPT = 78
EPT = CHUNK * CPT
N_EXTRA = (N_EDGES - EPT * NW) // CHUNK
EXTRA_BASE = EPT * NW

NBUF = 3
LOOKAHEAD = 2
NI = 6
ILOOK = 3

STRIPE = 640
LAST_STRIPE = N_NODES - 15 * STRIPE

_mesh = plsc.VectorSubcoreMesh(core_axis_name="c", subcore_axis_name="s")


@functools.partial(
    pl.kernel,
    mesh=_mesh,
    out_type=jax.ShapeDtypeStruct((NC, N_NODES, D), jnp.float32),
    scratch_types=[
        pltpu.VMEM((NI, 2, CHUNK), jnp.int32),
        pltpu.VMEM((2, CHUNK), jnp.int32),
        pltpu.VMEM((NBUF, CHUNK, D), jnp.float32),
        pltpu.VMEM_SHARED((N_NODES, D), jnp.float32),
        pltpu.SemaphoreType.DMA((NI,)),
        pltpu.SemaphoreType.DMA,
        pltpu.SemaphoreType.DMA((NBUF,)),
        pltpu.SemaphoreType.DMA((NBUF,)),
    ],
)
def _sc_aggregate(feat_hbm, edge_hbm, out_hbm, ibuf, xbuf, rows, acc,
                  isem, xsem, gsem, asem):
    c = lax.axis_index("c")
    s = lax.axis_index("s")
    wid = s * NC + c

    @pl.when(wid < N_EXTRA)
    def _():
        base = EXTRA_BASE + wid * CHUNK
        pltpu.async_copy(edge_hbm.at[0, pl.ds(base, CHUNK)], xbuf.at[0], xsem)
        pltpu.async_copy(edge_hbm.at[1, pl.ds(base, CHUNK)], xbuf.at[1], xsem)

    @pl.when(s < 15)
    def _():
        sl = pl.ds(s * STRIPE, STRIPE)
        pltpu.sync_copy(feat_hbm.at[sl], acc.at[sl])

    @pl.when(s == 15)
    def _():
        sl = pl.ds(15 * STRIPE, LAST_STRIPE)
        pltpu.sync_copy(feat_hbm.at[sl], acc.at[sl])

    plsc.subcore_barrier()

    def ifetch(j):
        base = wid * EPT + j * CHUNK
        h0 = pltpu.async_copy(
            edge_hbm.at[0, pl.ds(base, CHUNK)], ibuf.at[j % NI, 0],
            isem.at[j % NI])
        h1 = pltpu.async_copy(
            edge_hbm.at[1, pl.ds(base, CHUNK)], ibuf.at[j % NI, 1],
            isem.at[j % NI])
        return h0, h1

    def gather_start(j):
        return pltpu.async_copy(
            feat_hbm.at[ibuf.at[j % NI, 0]], rows.at[j % NBUF],
            gsem.at[j % NBUF])

    def add_start(j):
        return pltpu.async_copy(
            rows.at[j % NBUF], acc.at[ibuf.at[j % NI, 1]],
            asem.at[j % NBUF], add=True)

    ih, gh, ah = {}, {}, {}
    for j in range(ILOOK):
        ih[j] = ifetch(j)
    for j in range(LOOKAHEAD):
        ih[j][0].wait()
        ih[j][1].wait()
        gh[j] = gather_start(j)
    for j in range(CPT):
        ji = j + ILOOK
        if ji < CPT:
            ih[ji] = ifetch(ji)
        jn = j + LOOKAHEAD
        if jn < CPT:
            if jn - NBUF >= 0:
                ah[jn - NBUF].wait()
            ih[jn][0].wait()
            ih[jn][1].wait()
            gh[jn] = gather_start(jn)
        gh[j].wait()
        ah[j] = add_start(j)
    for j in range(CPT - NBUF, CPT):
        ah[j].wait()

    @pl.when(wid < N_EXTRA)
    def _():
        pltpu.make_async_copy(
            edge_hbm.at[0, pl.ds(EXTRA_BASE, CHUNK)], xbuf.at[0], xsem).wait()
        pltpu.make_async_copy(
            edge_hbm.at[1, pl.ds(EXTRA_BASE, CHUNK)], xbuf.at[1], xsem).wait()
        pltpu.async_copy(
            feat_hbm.at[xbuf.at[0]], rows.at[0], gsem.at[0]).wait()
        pltpu.async_copy(
            rows.at[0], acc.at[xbuf.at[1]], asem.at[0], add=True).wait()

    plsc.subcore_barrier()

    @pl.when(s < 15)
    def _():
        sl = pl.ds(s * STRIPE, STRIPE)
        pltpu.sync_copy(acc.at[sl], out_hbm.at[c, sl])

    @pl.when(s == 15)
    def _():
        sl = pl.ds(15 * STRIPE, LAST_STRIPE)
        pltpu.sync_copy(acc.at[sl], out_hbm.at[c, sl])


def _combine_body(p_ref, f_ref, o_ref):
    o_ref[...] = p_ref[0] + p_ref[1] - f_ref[...]


_ROWS_BLK = 5000

_combine = pl.pallas_call(
    _combine_body,
    grid=(N_NODES // _ROWS_BLK,),
    in_specs=[
        pl.BlockSpec((NC, _ROWS_BLK, D), lambda i: (0, i, 0)),
        pl.BlockSpec((_ROWS_BLK, D), lambda i: (i, 0)),
    ],
    out_specs=pl.BlockSpec((_ROWS_BLK, D), lambda i: (i, 0)),
    out_shape=jax.ShapeDtypeStruct((N_NODES, D), jnp.float32),
)


def kernel(feat, edge_index):
    partial = _sc_aggregate(feat, edge_index.astype(jnp.int32))
    return _combine(partial, feat)
